# baseline (speedup 1.0000x reference)
import functools

import jax
import jax.numpy as jnp
from jax import lax
from jax.experimental import pallas as pl
from jax.experimental.pallas import tpu as pltpu
from jax.experimental.pallas import tpu_sc as plsc

BATCH = 4096
SEQ = 200
D = 64
TOT = BATCH * SEQ
EPS = 1e-6

_info = plsc.get_sparse_core_info()
NC, NS = _info.num_cores, _info.num_subcores
NW = NC * NS
PW = TOT // NW
GROUP = 128
CH = 256
NCH = PW // CH
IDX_ROWS = PW // GROUP
POS_WORDS = SEQ * D


def _allsum(v, iota):
    for k in (8, 4, 2, 1):
        v = v + v.at[jnp.bitwise_xor(iota, k)].get(mode="promise_in_bounds")
    return v


def _body(tok_hbm, tab_hbm, pos_hbm, gb_hbm, out_hbm,
          idx_v, in_v, out_v, pos_v, gb_v, gsem, ssem):
    w = lax.axis_index("s") * NC + lax.axis_index("c")
    base = w * PW

    pltpu.sync_copy(tok_hbm.at[pl.ds(w * IDX_ROWS, IDX_ROWS)], idx_v)
    pltpu.sync_copy(pos_hbm.at[pl.ds(0, POS_WORDS)], pos_v)
    pltpu.sync_copy(gb_hbm, gb_v)

    g = [gb_v[pl.ds(i * 16, 16)] for i in range(4)]
    b = [gb_v[pl.ds(D + i * 16, 16)] for i in range(4)]
    iota = lax.iota(jnp.int32, 16)

    def chunk(c, _):
        for k in range(CH // GROUP):
            pltpu.async_copy(
                tab_hbm.at[idx_v.at[c * (CH // GROUP) + k]],
                in_v.at[pl.ds(k * GROUP, GROUP)],
                gsem,
            ).wait()

        def row(j, _):
            f = base + c * CH + j
            p = lax.rem(f, SEQ) * D
            x = [in_v[j, pl.ds(i * 16, 16)] + pos_v[pl.ds(p + i * 16, 16)]
                 for i in range(4)]
            s = _allsum((x[0] + x[1]) + (x[2] + x[3]), iota)
            ss = _allsum((x[0] * x[0] + x[1] * x[1])
                         + (x[2] * x[2] + x[3] * x[3]), iota)
            mean = s * (1.0 / D)
            var = ss * (1.0 / D) - mean * mean
            t = var + EPS
            iv = lax.bitcast_convert_type(t, jnp.int32)
            iv = 1597463007 - lax.shift_right_logical(iv, 1)
            y = lax.bitcast_convert_type(iv, jnp.float32)
            h = t * 0.5
            y = y * (1.5 - h * y * y)
            y = y * (1.5 - h * y * y)
            y = y * (1.5 - h * y * y)
            for i in range(4):
                out_v[j, pl.ds(i * 16, 16)] = (x[i] - mean) * (y * g[i]) + b[i]
            return 0

        lax.fori_loop(0, CH, row, 0)
        pltpu.sync_copy(out_v, out_hbm.at[pl.ds(base + c * CH, CH)])
        return 0

    lax.fori_loop(0, NCH, chunk, 0)


@functools.partial(jax.jit, static_argnames=())
def _run(tok2d, table, posflat, gb):
    mesh = plsc.VectorSubcoreMesh(core_axis_name="c", subcore_axis_name="s")
    f = functools.partial(
        pl.kernel,
        mesh=mesh,
        out_type=jax.ShapeDtypeStruct((TOT, D), jnp.float32),
        scratch_types=[
            pltpu.VMEM((NW * IDX_ROWS // NW, GROUP), jnp.int32),
            pltpu.VMEM((CH, D), jnp.float32),
            pltpu.VMEM((CH, D), jnp.float32),
            pltpu.VMEM((POS_WORDS,), jnp.float32),
            pltpu.VMEM((2 * D,), jnp.float32),
            pltpu.SemaphoreType.DMA,
            pltpu.SemaphoreType.DMA,
        ],
        compiler_params=pltpu.CompilerParams(use_tc_tiling_on_sc=False),
    )(_body)
    return f(tok2d, table, posflat, gb)


def kernel(inputs, token_table, pos_table, ln_gamma, ln_beta):
    tok2d = inputs.reshape(TOT // GROUP, GROUP).astype(jnp.int32)
    posflat = pos_table.reshape(-1)
    gb = jnp.concatenate([ln_gamma, ln_beta])
    out = _run(tok2d, token_table, posflat, gb)
    return out.reshape(BATCH, SEQ, D)


# speedup vs baseline: 1.1790x; 1.1790x over previous
import functools

import jax
import jax.numpy as jnp
from jax import lax
from jax.experimental import pallas as pl
from jax.experimental.pallas import tpu as pltpu
from jax.experimental.pallas import tpu_sc as plsc

BATCH = 4096
SEQ = 200
D = 64
TOT = BATCH * SEQ
EPS = 1e-6

_info = plsc.get_sparse_core_info()
NC, NS = _info.num_cores, _info.num_subcores
NW = NC * NS
PW = TOT // NW
GROUP = 128
CH = 256
NCH = PW // CH
IDX_ROWS = PW // GROUP
POS_WORDS = SEQ * D
U = 8


def _allsum(v, iota):
    for k in (8, 4, 2, 1):
        v = v + v.at[jnp.bitwise_xor(iota, k)].get(mode="promise_in_bounds")
    return v


def _body(tok_hbm, tab_hbm, pos_hbm, gb_hbm, out_hbm,
          idx_v, in_v, out_v, pos_v, gb_v, gsem0, gsem1, ssem0, ssem1):
    w = lax.axis_index("s") * NC + lax.axis_index("c")
    base = w * PW

    pltpu.sync_copy(tok_hbm.at[pl.ds(w * IDX_ROWS, IDX_ROWS)], idx_v)
    pltpu.sync_copy(pos_hbm.at[pl.ds(0, POS_WORDS)], pos_v)
    pltpu.sync_copy(gb_hbm, gb_v)

    g = [gb_v[pl.ds(i * 16, 16)] for i in range(4)]
    b = [gb_v[pl.ds(D + i * 16, 16)] for i in range(4)]
    iota = lax.iota(jnp.int32, 16)
    gsems = (gsem0, gsem1)
    ssems = (ssem0, ssem1)

    def issue_gathers(c, half):
        for k in range(CH // GROUP):
            pltpu.async_copy(
                tab_hbm.at[idx_v.at[c * (CH // GROUP) + k]],
                in_v.at[pl.ds(half * CH + k * GROUP, GROUP)],
                gsems[half],
            )

    def wait_gathers(c, half):
        for k in range(CH // GROUP):
            pltpu.make_async_copy(
                tab_hbm.at[idx_v.at[c * (CH // GROUP) + k]],
                in_v.at[pl.ds(half * CH + k * GROUP, GROUP)],
                gsems[half],
            ).wait()

    def store_desc(c, half):
        return pltpu.make_async_copy(
            out_v.at[pl.ds(half * CH, CH)],
            out_hbm.at[pl.ds(base + c * CH, CH)],
            ssems[half],
        )

    def compute(c, half):
        buf_off = half * CH
        p0 = lax.rem(c * CH, SEQ) * D

        def row_group(i, p):
            r0 = buf_off + i * U
            for u in range(U):
                r = r0 + u
                x = [in_v[r, pl.ds(i2 * 16, 16)]
                     + pos_v[pl.ds(p + i2 * 16, 16)] for i2 in range(4)]
                s = _allsum((x[0] + x[1]) + (x[2] + x[3]), iota)
                ss = _allsum((x[0] * x[0] + x[1] * x[1])
                             + (x[2] * x[2] + x[3] * x[3]), iota)
                mean = s * (1.0 / D)
                var = ss * (1.0 / D) - mean * mean
                t = var + EPS
                iv = lax.bitcast_convert_type(t, jnp.int32)
                iv = 1597463007 - lax.shift_right_logical(iv, 1)
                y = lax.bitcast_convert_type(iv, jnp.float32)
                h = t * 0.5
                y = y * (1.5 - h * y * y)
                y = y * (1.5 - h * y * y)
                for i2 in range(4):
                    out_v[r, pl.ds(i2 * 16, 16)] = (
                        (x[i2] - mean) * (y * g[i2]) + b[i2])
                p = p + D
                p = jnp.where(p == POS_WORDS, 0, p)
            return p

        lax.fori_loop(0, CH // U, row_group, p0)

    issue_gathers(0, 0)
    issue_gathers(1, 1)

    def pair(i, _):
        for half in range(2):
            c = 2 * i + half
            wait_gathers(c, half)

            @pl.when(i >= 1)
            def _():
                store_desc(c - 2, half).wait()

            compute(c, half)
            store_desc(c, half).start()

            @pl.when(c + 2 < NCH)
            def _():
                issue_gathers(c + 2, half)
        return 0

    lax.fori_loop(0, NCH // 2, pair, 0)
    store_desc(NCH - 2, 0).wait()
    store_desc(NCH - 1, 1).wait()


@jax.jit
def _run(tok2d, table, posflat, gb):
    mesh = plsc.VectorSubcoreMesh(core_axis_name="c", subcore_axis_name="s")
    f = functools.partial(
        pl.kernel,
        mesh=mesh,
        out_type=jax.ShapeDtypeStruct((TOT, D), jnp.float32),
        scratch_types=[
            pltpu.VMEM((IDX_ROWS, GROUP), jnp.int32),
            pltpu.VMEM((2 * CH, D), jnp.float32),
            pltpu.VMEM((2 * CH, D), jnp.float32),
            pltpu.VMEM((POS_WORDS,), jnp.float32),
            pltpu.VMEM((2 * D,), jnp.float32),
            pltpu.SemaphoreType.DMA,
            pltpu.SemaphoreType.DMA,
            pltpu.SemaphoreType.DMA,
            pltpu.SemaphoreType.DMA,
        ],
        compiler_params=pltpu.CompilerParams(use_tc_tiling_on_sc=False),
    )(_body)
    return f(tok2d, table, posflat, gb)


def kernel(inputs, token_table, pos_table, ln_gamma, ln_beta):
    tok2d = inputs.reshape(TOT // GROUP, GROUP).astype(jnp.int32)
    posflat = pos_table.reshape(-1)
    gb = jnp.concatenate([ln_gamma, ln_beta])
    out = _run(tok2d, token_table, posflat, gb)
    return out.reshape(BATCH, SEQ, D)
